# baseline (speedup 1.0000x reference)
import jax
import jax.numpy as jnp
from jax import lax
from jax.experimental import pallas as pl
from jax.experimental.pallas import tpu as pltpu
from jax.experimental.pallas import tpu_sc as plsc

_N = 100000
_D = 3
_G = 1024
_NC = 2
_NS = 16
_NT = _NC * _NS
_ROWS = 3200
_ACC = 1152
_NCH = _ACC // 128
_OVL = _NT * _ROWS - _N
_LAST_START = _N - _ROWS


def _stage1_body(a_hbm, b_hbm, bp_hbm, id_hbm, sums_hbm, counts_hbm,
                 a_v, b_v, idx_v, ident_v, acc_s, acc_c, zero_v, chunk_v,
                 sums_sh, counts_sh, sem_in, sem_sc):
    c = lax.axis_index("c")
    s = lax.axis_index("s")
    wid = c * _NS + s

    ones16 = jnp.ones((16,), jnp.float32)
    zeros16 = jnp.zeros((16,), jnp.float32)

    start = jnp.where(wid == _NT - 1, _LAST_START, wid * _ROWS)
    off = pl.multiple_of(start, 8)
    descs = [pltpu.async_copy(bp_hbm.at[pl.ds(pl.multiple_of(wid * _ROWS, 8),
                                              _ROWS)], idx_v, sem_in),
             pltpu.async_copy(id_hbm, ident_v, sem_in)]
    for k in range(_D):
        descs.append(
            pltpu.async_copy(a_hbm.at[pl.ds(off + k * _N, _ROWS)],
                             a_v.at[pl.ds(k * _ROWS, _ROWS)], sem_in))
        descs.append(
            pltpu.async_copy(b_hbm.at[pl.ds(off + k * _N, _ROWS)],
                             b_v.at[pl.ds(k * _ROWS, _ROWS)], sem_in))

    @plsc.parallel_loop(0, _ACC, 16, unroll=8)
    def _zero_local(i):
        i0 = pl.multiple_of(i, 16)
        acc_s[pl.ds(i0, 16)] = zeros16
        acc_c[pl.ds(i0, 16)] = zeros16

    @pl.when(s == 0)
    def _zero_spmem():
        @plsc.parallel_loop(0, _ACC, 16, unroll=8)
        def _fz(i):
            zero_v[pl.ds(pl.multiple_of(i, 16), 16)] = zeros16
        pltpu.sync_copy(zero_v, sums_sh)
        pltpu.sync_copy(zero_v, counts_sh)

    for d in descs:
        d.wait()

    def _acc16(i, carry):
        i0 = pl.multiple_of(i * 16, 16)
        g0 = a_v[pl.ds(i0, 16)]
        g1 = a_v[pl.ds(i0 + _ROWS, 16)]
        g2 = a_v[pl.ds(i0 + 2 * _ROWS, 16)]
        h0 = b_v[pl.ds(i0, 16)]
        h1 = b_v[pl.ds(i0 + _ROWS, 16)]
        h2 = b_v[pl.ds(i0 + 2 * _ROWS, 16)]
        e = jnp.abs(g0 - h0) + jnp.abs(g1 - h1) + jnp.abs(g2 - h2)
        idx16 = idx_v[pl.ds(i0, 16)]
        plsc.addupdate_scatter(acc_s, [idx16], e)
        plsc.addupdate_scatter(acc_c, [idx16], ones16)
        return carry
    lax.fori_loop(0, _ROWS // 16, _acc16, 0)

    plsc.subcore_barrier()

    descs = []
    for j in range(_NCH):
        descs.append(pltpu.async_copy(acc_s.at[pl.ds(j * 128, 128)],
                                      sums_sh.at[ident_v.at[j]], sem_sc,
                                      add=True))
        descs.append(pltpu.async_copy(acc_c.at[pl.ds(j * 128, 128)],
                                      counts_sh.at[ident_v.at[j]], sem_sc,
                                      add=True))
    for d in descs:
        d.wait()

    plsc.subcore_barrier()

    @pl.when(s < 8)
    def _writeout():
        off128 = pl.multiple_of(s * 128, 8)
        pltpu.sync_copy(sums_sh.at[pl.ds(off128, 128)], chunk_v)
        pltpu.sync_copy(chunk_v, sums_hbm.at[c, s])
        pltpu.sync_copy(counts_sh.at[pl.ds(off128, 128)], chunk_v)
        pltpu.sync_copy(chunk_v, counts_hbm.at[c, s])


_stage1 = pl.kernel(
    _stage1_body,
    out_type=(jax.ShapeDtypeStruct((_NC, 8, 128), jnp.float32),
              jax.ShapeDtypeStruct((_NC, 8, 128), jnp.float32)),
    mesh=plsc.VectorSubcoreMesh(core_axis_name="c", subcore_axis_name="s"),
    compiler_params=pltpu.CompilerParams(needs_layout_passes=False),
    scratch_types=(
        pltpu.VMEM((_ROWS * _D,), jnp.float32),
        pltpu.VMEM((_ROWS * _D,), jnp.float32),
        pltpu.VMEM((_ROWS,), jnp.int32),
        pltpu.VMEM((_NCH, 128), jnp.int32),
        pltpu.VMEM((_ACC,), jnp.float32),
        pltpu.VMEM((_ACC,), jnp.float32),
        pltpu.VMEM((_ACC,), jnp.float32),
        pltpu.VMEM((128,), jnp.float32),
        pltpu.VMEM_SHARED((_ACC,), jnp.float32),
        pltpu.VMEM_SHARED((_ACC,), jnp.float32),
        pltpu.SemaphoreType.DMA,
        pltpu.SemaphoreType.DMA,
    ),
)


def _stage2_body(sums_ref, counts_ref, err_ref, loss_ref):
    ssum = sums_ref[0] + sums_ref[1]
    cnt = counts_ref[0] + counts_ref[1]
    err_ref[...] = jnp.where(cnt > 0.0,
                             ssum / (3.0 * jnp.maximum(cnt, 1.0)),
                             0.0)
    loss_ref[...] = jnp.sum(ssum, keepdims=True).reshape(1, 1) * (1.0 / (_N * _D))


def kernel(input, target, batch):
    a = input.T.reshape(-1)
    b = target.T.reshape(-1)
    bi = batch.astype(jnp.int32)
    bp = jnp.concatenate([
        bi[:_LAST_START + _OVL],
        jnp.full((_OVL,), _G, jnp.int32),
        bi[_LAST_START + _OVL:],
    ])
    ident = jnp.arange(_ACC, dtype=jnp.int32).reshape(_NCH, 128)

    sums, counts = _stage1(a, b, bp, ident)

    err2d, loss2d = pl.pallas_call(
        _stage2_body,
        out_shape=(jax.ShapeDtypeStruct((8, 128), jnp.float32),
                   jax.ShapeDtypeStruct((1, 1), jnp.float32)),
    )(sums, counts)

    return (loss2d[0, 0], err2d.reshape(_G))


# speedup vs baseline: 1.1355x; 1.1355x over previous
import jax
import jax.numpy as jnp
from jax import lax
from jax.experimental import pallas as pl
from jax.experimental.pallas import tpu as pltpu
from jax.experimental.pallas import tpu_sc as plsc

_N = 100000
_D = 3
_G = 1024
_NC = 2
_NS = 16
_NT = _NC * _NS
_ROWS = 3200
_NCH = _ROWS // 128
_ACC = 1040
_OVL = _NT * _ROWS - _N
_LAST_START = _N - _ROWS


def _stage1_body(a_hbm, b_hbm, bp_hbm, sums_hbm, counts_hbm,
                 a_v, b_v, idx_v, rm_v, ones_v, zero_v, chunk_v,
                 sums_sh, counts_sh, sem_in, sem_sc):
    c = lax.axis_index("c")
    s = lax.axis_index("s")
    wid = c * _NS + s

    ones16 = jnp.ones((16,), jnp.float32)
    zeros16 = jnp.zeros((16,), jnp.float32)

    start = jnp.where(wid == _NT - 1, _LAST_START, wid * _ROWS)
    off = pl.multiple_of(start, 8)
    descs = [pltpu.async_copy(bp_hbm.at[wid], idx_v, sem_in)]
    for k in range(_D):
        descs.append(
            pltpu.async_copy(a_hbm.at[pl.ds(off + k * _N, _ROWS)],
                             a_v.at[pl.ds(k * _ROWS, _ROWS)], sem_in))
        descs.append(
            pltpu.async_copy(b_hbm.at[pl.ds(off + k * _N, _ROWS)],
                             b_v.at[pl.ds(k * _ROWS, _ROWS)], sem_in))

    @plsc.parallel_loop(0, _ROWS, 16, unroll=8)
    def _fill_ones(i):
        ones_v[pl.ds(pl.multiple_of(i, 16), 16)] = ones16

    @pl.when(s == 0)
    def _zero_spmem():
        @plsc.parallel_loop(0, _ACC, 16, unroll=8)
        def _fz(i):
            zero_v[pl.ds(pl.multiple_of(i, 16), 16)] = zeros16
        pltpu.sync_copy(zero_v, sums_sh)
        pltpu.sync_copy(zero_v, counts_sh)

    for d in descs:
        d.wait()

    @plsc.parallel_loop(0, _ROWS, 16, unroll=4)
    def _body(i):
        i0 = pl.multiple_of(i, 16)
        g0 = a_v[pl.ds(i0, 16)]
        g1 = a_v[pl.ds(i0 + _ROWS, 16)]
        g2 = a_v[pl.ds(i0 + 2 * _ROWS, 16)]
        h0 = b_v[pl.ds(i0, 16)]
        h1 = b_v[pl.ds(i0 + _ROWS, 16)]
        h2 = b_v[pl.ds(i0 + 2 * _ROWS, 16)]
        e = jnp.abs(g0 - h0) + jnp.abs(g1 - h1) + jnp.abs(g2 - h2)
        rm_v[pl.ds(i0, 16)] = e

    plsc.subcore_barrier()

    descs = []
    for j in range(_NCH):
        descs.append(pltpu.async_copy(rm_v.at[pl.ds(j * 128, 128)],
                                      sums_sh.at[idx_v.at[j]], sem_sc,
                                      add=True))
        descs.append(pltpu.async_copy(ones_v.at[pl.ds(j * 128, 128)],
                                      counts_sh.at[idx_v.at[j]], sem_sc,
                                      add=True))
    for d in descs:
        d.wait()

    plsc.subcore_barrier()

    @pl.when(s < 8)
    def _writeout():
        off128 = pl.multiple_of(s * 128, 8)
        pltpu.sync_copy(sums_sh.at[pl.ds(off128, 128)], chunk_v)
        pltpu.sync_copy(chunk_v, sums_hbm.at[c, s])
        pltpu.sync_copy(counts_sh.at[pl.ds(off128, 128)], chunk_v)
        pltpu.sync_copy(chunk_v, counts_hbm.at[c, s])


_stage1 = pl.kernel(
    _stage1_body,
    out_type=(jax.ShapeDtypeStruct((_NC, 8, 128), jnp.float32),
              jax.ShapeDtypeStruct((_NC, 8, 128), jnp.float32)),
    mesh=plsc.VectorSubcoreMesh(core_axis_name="c", subcore_axis_name="s"),
    compiler_params=pltpu.CompilerParams(needs_layout_passes=False),
    scratch_types=(
        pltpu.VMEM((_ROWS * _D,), jnp.float32),
        pltpu.VMEM((_ROWS * _D,), jnp.float32),
        pltpu.VMEM((_NCH, 128), jnp.int32),
        pltpu.VMEM((_ROWS,), jnp.float32),
        pltpu.VMEM((_ROWS,), jnp.float32),
        pltpu.VMEM((_ACC,), jnp.float32),
        pltpu.VMEM((128,), jnp.float32),
        pltpu.VMEM_SHARED((_ACC,), jnp.float32),
        pltpu.VMEM_SHARED((_ACC,), jnp.float32),
        pltpu.SemaphoreType.DMA,
        pltpu.SemaphoreType.DMA,
    ),
)


def _stage2_body(sums_ref, counts_ref, err_ref, loss_ref):
    ssum = sums_ref[0] + sums_ref[1]
    cnt = counts_ref[0] + counts_ref[1]
    err_ref[...] = jnp.where(cnt > 0.0,
                             ssum / (3.0 * jnp.maximum(cnt, 1.0)),
                             0.0)
    loss_ref[...] = jnp.sum(ssum, keepdims=True).reshape(1, 1) * (1.0 / (_N * _D))


def kernel(input, target, batch):
    a = input.T.reshape(-1)
    b = target.T.reshape(-1)
    bi = batch.astype(jnp.int32)
    bp = jnp.concatenate([
        bi[:_LAST_START + _OVL],
        jnp.full((_OVL,), _G, jnp.int32),
        bi[_LAST_START + _OVL:],
    ]).reshape(_NT, _NCH, 128)

    sums, counts = _stage1(a, b, bp)

    err2d, loss2d = pl.pallas_call(
        _stage2_body,
        out_shape=(jax.ShapeDtypeStruct((8, 128), jnp.float32),
                   jax.ShapeDtypeStruct((1, 1), jnp.float32)),
    )(sums, counts)

    return (loss2d[0, 0], err2d.reshape(_G))
